# baseline (speedup 1.0000x reference)
import functools

import jax
import jax.numpy as jnp
from jax import lax
from jax.experimental import pallas as pl
from jax.experimental.pallas import tpu as pltpu
from jax.experimental.pallas import tpu_sc as plsc

NUM_FIELDS = 26
BATCH = 16384
EMBED_DIM = 32

NUM_WORKERS = 32
STREAM_LEN = 4 * NUM_FIELDS
NUM_STREAMS = BATCH * NUM_FIELDS // STREAM_LEN
STREAMS_PER_W = NUM_STREAMS // NUM_WORKERS
STREAMS_PER_CHUNK = 8
NUM_CHUNKS = STREAMS_PER_W // STREAMS_PER_CHUNK
NBUF = 2


def _make_gather():
  mesh = plsc.VectorSubcoreMesh(core_axis_name="c", subcore_axis_name="s")

  @functools.partial(
      pl.kernel,
      mesh=mesh,
      out_type=jax.ShapeDtypeStruct((NUM_STREAMS, STREAM_LEN, EMBED_DIM),
                                    jnp.float32),
      scratch_types=[
          pltpu.VMEM((NBUF, STREAMS_PER_CHUNK, STREAM_LEN), jnp.int32),
          pltpu.VMEM((NBUF, STREAMS_PER_CHUNK, STREAM_LEN, EMBED_DIM),
                     jnp.float32),
          pltpu.SemaphoreType.DMA,
          pltpu.SemaphoreType.DMA,
      ],
      compiler_params=pltpu.CompilerParams(use_tc_tiling_on_sc=False),
  )
  def gather_kernel(table_hbm, x2, out3, idx_v, rows_v, sem0, sem1):
    sems = (sem0, sem1)
    wid = lax.axis_index("s") * 2 + lax.axis_index("c")
    base = wid * STREAMS_PER_W

    def fire(g, b):
      pltpu.sync_copy(
          x2.at[pl.ds(base + g * STREAMS_PER_CHUNK, STREAMS_PER_CHUNK)],
          idx_v.at[b])
      for j in range(STREAMS_PER_CHUNK):
        pltpu.async_copy(
            table_hbm.at[idx_v.at[b].at[j]],
            rows_v.at[b].at[j],
            sems[b])

    def drain_and_writeback(g, b):
      pltpu.make_async_copy(
          out3.at[pl.ds(0, STREAMS_PER_CHUNK)], rows_v.at[b], sems[b]).wait()
      pltpu.sync_copy(
          rows_v.at[b],
          out3.at[pl.ds(base + g * STREAMS_PER_CHUNK, STREAMS_PER_CHUNK)])

    fire(0, 0)
    fire(1, 1)

    def body(k, _):
      for b in range(NBUF):
        g = NBUF * k + b
        drain_and_writeback(g, b)

        @pl.when(g + NBUF < NUM_CHUNKS)
        def _():
          fire(g + NBUF, b)
      return ()

    lax.fori_loop(0, NUM_CHUNKS // NBUF, body, (), unroll=False)

  return gather_kernel


_gather = _make_gather()


@jax.jit
def kernel(x, embs):
  x2 = x.reshape(NUM_STREAMS, STREAM_LEN)
  out3 = _gather(embs, x2)
  return out3.reshape(BATCH, NUM_FIELDS * EMBED_DIM)


# speedup vs baseline: 1.2499x; 1.2499x over previous
import functools

import jax
import jax.numpy as jnp
from jax import lax
from jax.experimental import pallas as pl
from jax.experimental.pallas import tpu as pltpu
from jax.experimental.pallas import tpu_sc as plsc

NUM_FIELDS = 26
BATCH = 16384
EMBED_DIM = 32

NUM_WORKERS = 32
STREAM_LEN = 4 * NUM_FIELDS
NUM_STREAMS = BATCH * NUM_FIELDS // STREAM_LEN
STREAMS_PER_W = NUM_STREAMS // NUM_WORKERS
STREAMS_PER_CHUNK = 8
NUM_CHUNKS = STREAMS_PER_W // STREAMS_PER_CHUNK
NBUF = 2


def _make_gather():
  mesh = plsc.VectorSubcoreMesh(core_axis_name="c", subcore_axis_name="s")

  @functools.partial(
      pl.kernel,
      mesh=mesh,
      out_type=jax.ShapeDtypeStruct((NUM_STREAMS, STREAM_LEN, EMBED_DIM),
                                    jnp.float32),
      scratch_types=[
          pltpu.VMEM((NBUF, STREAMS_PER_CHUNK, STREAM_LEN), jnp.int32),
          pltpu.VMEM((NBUF, STREAMS_PER_CHUNK, STREAM_LEN, EMBED_DIM),
                     jnp.float32),
          pltpu.SemaphoreType.DMA,
          pltpu.SemaphoreType.DMA,
      ],
      compiler_params=pltpu.CompilerParams(use_tc_tiling_on_sc=False),
  )
  def gather_kernel(table_hbm, x2, out3, idx_v, rows_v, sem0, sem1):
    sems = (sem0, sem1)
    wid = lax.axis_index("s") * 2 + lax.axis_index("c")
    base = wid * STREAMS_PER_W

    def fire(g, b):
      pltpu.sync_copy(
          x2.at[pl.ds(base + g * STREAMS_PER_CHUNK, STREAMS_PER_CHUNK)],
          idx_v.at[b])
      for j in range(STREAMS_PER_CHUNK):
        pltpu.async_copy(
            table_hbm.at[idx_v.at[b].at[j]],
            rows_v.at[b].at[j],
            sems[b])

    def drain_and_writeback(g, b):
      pltpu.make_async_copy(
          out3.at[pl.ds(0, STREAMS_PER_CHUNK)], rows_v.at[b], sems[b]).wait()
      pltpu.sync_copy(
          rows_v.at[b],
          out3.at[pl.ds(base + g * STREAMS_PER_CHUNK, STREAMS_PER_CHUNK)])

    fire(0, 0)
    fire(1, 1)

    def body(k, _):
      for b in range(NBUF):
        g = NBUF * k + b
        drain_and_writeback(g, b)

        @pl.when(g + NBUF < NUM_CHUNKS)
        def _():
          fire(g + NBUF, b)
      return ()

    lax.fori_loop(0, NUM_CHUNKS // NBUF, body, (), unroll=False)

  return gather_kernel


_gather = _make_gather()

VOCAB = 1000000
TBLK = 12800
TGRID = -(-VOCAB // TBLK)
TOUT = TBLK * EMBED_DIM // 128


def _transpose_body(in_ref, out_ref):
  t = in_ref[...].T.reshape(TOUT, 4, EMBED_DIM)
  for c in range(4):
    out_ref[:, 32 * c:32 * (c + 1)] = t[:, c, :]


_transpose = pl.pallas_call(
    _transpose_body,
    grid=(TGRID,),
    in_specs=[pl.BlockSpec((EMBED_DIM, TBLK), lambda i: (0, i))],
    out_specs=pl.BlockSpec((TOUT, 128), lambda i: (i, 0)),
    out_shape=jax.ShapeDtypeStruct((VOCAB * EMBED_DIM // 128, 128),
                                   jnp.float32),
)


@jax.jit
def kernel(x, embs):
  table_lin = _transpose(embs.T).reshape(VOCAB, EMBED_DIM)
  x2 = x.reshape(NUM_STREAMS, STREAM_LEN)
  out3 = _gather(table_lin, x2)
  return out3.reshape(BATCH, NUM_FIELDS * EMBED_DIM)


# speedup vs baseline: 2.2652x; 1.8124x over previous
import functools

import jax
import jax.numpy as jnp
from jax import lax
from jax.experimental import pallas as pl
from jax.experimental.pallas import tpu as pltpu
from jax.experimental.pallas import tpu_sc as plsc

NUM_FIELDS = 26
BATCH = 16384
EMBED_DIM = 32
VOCAB = 1000000

TBLK = 12800
TGRID = -(-VOCAB // TBLK)
TOUT = TBLK * EMBED_DIM // 128
VROWS = TGRID * TBLK
OUT_ROWS = 250048
VOCAB_PAD = OUT_ROWS * 4


def _transpose_body(in_ref, out_ref):
  for s in range(TBLK // 512):
    stacked = jnp.concatenate(
        [in_ref[:, 512 * s + 128 * c:512 * s + 128 * (c + 1)]
         for c in range(4)], axis=0)
    out_ref[128 * s:128 * (s + 1), :] = stacked.T


_transpose = pl.pallas_call(
    _transpose_body,
    grid=(TGRID,),
    in_specs=[pl.BlockSpec((EMBED_DIM, TBLK), lambda i: (0, i))],
    out_specs=pl.BlockSpec((TOUT, 128), lambda i: (i, 0)),
    out_shape=jax.ShapeDtypeStruct((OUT_ROWS, 128), jnp.float32),
)

NUM_WORKERS = 32
TOTAL = BATCH * NUM_FIELDS
PER_WORKER = TOTAL // NUM_WORKERS
STREAM_LEN = 104
STREAMS_PER_CHUNK = 8
CHUNK = STREAM_LEN * STREAMS_PER_CHUNK
NUM_CHUNKS = PER_WORKER // CHUNK
NBUF = 2
L = 16


def _make_gather():
  mesh = plsc.VectorSubcoreMesh(core_axis_name="c", subcore_axis_name="s")

  @functools.partial(
      pl.kernel,
      mesh=mesh,
      out_type=jax.ShapeDtypeStruct((TOTAL // STREAM_LEN, STREAM_LEN,
                                     EMBED_DIM), jnp.float32),
      scratch_types=[
          pltpu.VMEM((NBUF, CHUNK), jnp.int32),
          pltpu.VMEM((NBUF, STREAMS_PER_CHUNK, STREAM_LEN, EMBED_DIM),
                     jnp.float32),
          pltpu.SemaphoreType.DMA,
          pltpu.SemaphoreType.DMA,
      ],
      compiler_params=pltpu.CompilerParams(use_tc_tiling_on_sc=False),
  )
  def gather_kernel(table_hbm, x_hbm, out_hbm, idx_v, rows_v, sem0, sem1):
    sems = (sem0, sem1)
    wid = lax.axis_index("s") * 2 + lax.axis_index("c")
    base = wid * PER_WORKER

    def fire(g, b):
      pltpu.sync_copy(x_hbm.at[pl.ds(base + g * CHUNK, CHUNK)], idx_v.at[b])
      for k in range(CHUNK // L):
        i = idx_v[b, pl.ds(k * L, L)]
        pi = ((i & jnp.int32(~511)) | ((i & jnp.int32(127)) << 2)
              | ((i >> 7) & jnp.int32(3)))
        idx_v[b, pl.ds(k * L, L)] = pi
      for j in range(STREAMS_PER_CHUNK):
        pltpu.async_copy(
            table_hbm.at[idx_v.at[b].at[pl.ds(j * STREAM_LEN, STREAM_LEN)]],
            rows_v.at[b].at[j],
            sems[b])

    def drain_and_writeback(g, b):
      pltpu.make_async_copy(
          out_hbm.at[pl.ds(0, STREAMS_PER_CHUNK)], rows_v.at[b],
          sems[b]).wait()
      pltpu.sync_copy(
          rows_v.at[b],
          out_hbm.at[pl.ds((base + g * CHUNK) // STREAM_LEN,
                           STREAMS_PER_CHUNK)])

    fire(0, 0)
    fire(1, 1)

    def body(k, _):
      for b in range(NBUF):
        g = NBUF * k + b
        drain_and_writeback(g, b)

        @pl.when(g + NBUF < NUM_CHUNKS)
        def _():
          fire(g + NBUF, b)
      return ()

    lax.fori_loop(0, NUM_CHUNKS // NBUF, body, (), unroll=False)

  return gather_kernel


_gather = _make_gather()


@jax.jit
def kernel(x, embs):
  table = _transpose(embs.T).reshape(VOCAB_PAD, EMBED_DIM)
  out3 = _gather(table, x.reshape(-1))
  return out3.reshape(BATCH, NUM_FIELDS * EMBED_DIM)
